# Initial kernel scaffold; baseline (speedup 1.0000x reference)
import jax
import jax.numpy as jnp
from jax.experimental import pallas as pl


def kernel(inputs, T0, T1, W1, b1, T2, W2, b2):
    raise NotImplementedError("write your pallas kernel here")


# speedup vs baseline: 1.7520x; 1.7520x over previous
import functools

import jax
import jax.numpy as jnp
from jax import lax
from jax.experimental import pallas as pl
from jax.experimental.pallas import tpu as pltpu
from jax.experimental.pallas import tpu_sc as plsc

BATCH = 16384
BASE_DIM = 64
D1 = 32
D2 = 16
OFF1 = 100000
OFF2 = 400000

NUM_CORES = 2
NUM_SUBCORES = 16
NW = NUM_CORES * NUM_SUBCORES
BPW = BATCH // NW
CHUNK = 128
NCHUNK = BPW // CHUNK
LANES = 16


def _gather_body(ids_hbm, t0_hbm, t1_hbm, t2_hbm, e0_hbm, e1_hbm, e2_hbm,
                 ids_v, loc0_v, loc1_v, loc2_v, r0_v, r1_v, r2_v, gsem, wsem):
    wid = lax.axis_index("s") * NUM_CORES + lax.axis_index("c")
    base = wid * BPW
    pltpu.sync_copy(ids_hbm.at[pl.ds(base, BPW)], ids_v)
    for j in range(NCHUNK):
        for t in range(CHUNK // LANES):
            off = j * CHUNK + t * LANES
            ids16 = ids_v[pl.ds(off, LANES)]
            lt1 = ids16 < OFF1
            lt2 = ids16 < OFF2
            zero = jnp.zeros((LANES,), jnp.int32)
            loc0 = jnp.where(lt1, ids16 + 1, zero)
            loc1 = jnp.where(lt1, zero, jnp.where(lt2, ids16 - (OFF1 - 1), zero))
            loc2 = jnp.where(lt2, zero, ids16 - (OFF2 - 1))
            sl = pl.ds(t * LANES, LANES)
            loc0_v[j, sl] = loc0
            loc1_v[j, sl] = loc1
            loc2_v[j, sl] = loc2
    cps = []
    for j in range(NCHUNK):
        dst = pl.ds(j * CHUNK, CHUNK)
        cps.append(pltpu.async_copy(t0_hbm.at[loc0_v.at[j]], r0_v.at[dst], gsem))
        cps.append(pltpu.async_copy(t1_hbm.at[loc1_v.at[j]], r1_v.at[dst], gsem))
        cps.append(pltpu.async_copy(t2_hbm.at[loc2_v.at[j]], r2_v.at[dst], gsem))
    for cp in cps:
        cp.wait()
    out_sl = pl.ds(base, BPW)
    wps = [
        pltpu.async_copy(r0_v, e0_hbm.at[out_sl], wsem),
        pltpu.async_copy(r1_v, e1_hbm.at[out_sl], wsem),
        pltpu.async_copy(r2_v, e2_hbm.at[out_sl], wsem),
    ]
    for wp in wps:
        wp.wait()


def _sc_gather(ids, T0, T1, T2):
    mesh = plsc.VectorSubcoreMesh(
        core_axis_name="c", subcore_axis_name="s",
        num_cores=NUM_CORES, num_subcores=NUM_SUBCORES)
    f = pl.kernel(
        _gather_body,
        out_type=(
            jax.ShapeDtypeStruct((BATCH, BASE_DIM), jnp.float32),
            jax.ShapeDtypeStruct((BATCH, D1), jnp.float32),
            jax.ShapeDtypeStruct((BATCH, D2), jnp.float32),
        ),
        mesh=mesh,
        compiler_params=pltpu.CompilerParams(use_tc_tiling_on_sc=False),
        scratch_types=[
            pltpu.VMEM((BPW,), jnp.int32),
            pltpu.VMEM((NCHUNK, CHUNK), jnp.int32),
            pltpu.VMEM((NCHUNK, CHUNK), jnp.int32),
            pltpu.VMEM((NCHUNK, CHUNK), jnp.int32),
            pltpu.VMEM((BPW, BASE_DIM), jnp.float32),
            pltpu.VMEM((BPW, D1), jnp.float32),
            pltpu.VMEM((BPW, D2), jnp.float32),
            pltpu.SemaphoreType.DMA,
            pltpu.SemaphoreType.DMA,
        ],
    )
    return f(ids, T0, T1, T2)


BT = 2048


def _proj_body(ids_ref, e0_ref, e1_ref, e2_ref, w1_ref, b1_ref, w2_ref, b2_ref,
               out_ref):
    ids = ids_ref[...]
    m0 = (ids < OFF1).astype(jnp.float32)
    m2 = (ids >= OFF2).astype(jnp.float32)
    m1 = (1.0 - m0) - m2
    p1 = jnp.dot(e1_ref[...], w1_ref[...],
                 preferred_element_type=jnp.float32) + b1_ref[...]
    p2 = jnp.dot(e2_ref[...], w2_ref[...],
                 preferred_element_type=jnp.float32) + b2_ref[...]
    out_ref[...] = e0_ref[...] * m0 + p1 * m1 + p2 * m2


def _tc_project(ids2d, e0, e1, e2, W1, b1, W2, b2):
    grid = (BATCH // BT,)
    return pl.pallas_call(
        _proj_body,
        grid=grid,
        in_specs=[
            pl.BlockSpec((BT, 1), lambda i: (i, 0)),
            pl.BlockSpec((BT, BASE_DIM), lambda i: (i, 0)),
            pl.BlockSpec((BT, D1), lambda i: (i, 0)),
            pl.BlockSpec((BT, D2), lambda i: (i, 0)),
            pl.BlockSpec((D1, BASE_DIM), lambda i: (0, 0)),
            pl.BlockSpec((1, BASE_DIM), lambda i: (0, 0)),
            pl.BlockSpec((D2, BASE_DIM), lambda i: (0, 0)),
            pl.BlockSpec((1, BASE_DIM), lambda i: (0, 0)),
        ],
        out_specs=pl.BlockSpec((BT, BASE_DIM), lambda i: (i, 0)),
        out_shape=jax.ShapeDtypeStruct((BATCH, BASE_DIM), jnp.float32),
    )(ids2d, e0, e1, e2, W1, b1, W2, b2)


def kernel(inputs, T0, T1, W1, b1, T2, W2, b2):
    ids = inputs.astype(jnp.int32)
    e0, e1, e2 = _sc_gather(ids, T0, T1, T2)
    return _tc_project(ids.reshape(BATCH, 1), e0, e1, e2,
                       W1, b1.reshape(1, BASE_DIM), W2, b2.reshape(1, BASE_DIM))


# speedup vs baseline: 2.8385x; 1.6201x over previous
import functools

import jax
import jax.numpy as jnp
from jax import lax
from jax.experimental import pallas as pl
from jax.experimental.pallas import tpu as pltpu
from jax.experimental.pallas import tpu_sc as plsc

BATCH = 16384
BASE_DIM = 64
D1 = 32
D2 = 16
OFF1 = 100000
OFF2 = 400000

NUM_CORES = 2
NUM_SUBCORES = 16
NW = NUM_CORES * NUM_SUBCORES
BPW = BATCH // NW
CHUNK = 128
NCHUNK = BPW // CHUNK
LANES = 16


def _gather_body(ids_hbm, t0_hbm, t1_hbm, t2_hbm, e0_hbm, e1_hbm, e2_hbm,
                 ids_v, loc0_v, loc1_v, loc2_v, r0_v, r1_v, r2_v, gsem, wsem):
    wid = lax.axis_index("s") * NUM_CORES + lax.axis_index("c")
    base = wid * BPW
    pltpu.sync_copy(ids_hbm.at[pl.ds(base, BPW)], ids_v)
    for j in range(NCHUNK):
        for t in range(CHUNK // LANES):
            off = j * CHUNK + t * LANES
            ids16 = ids_v[pl.ds(off, LANES)]
            lt1 = ids16 < OFF1
            lt2 = ids16 < OFF2
            loc0 = jnp.where(lt1, ids16 + 1, (ids16 >> 4) + 1)
            dummy1 = (ids16 >> 2) + 1
            loc1 = jnp.where(lt1, dummy1,
                             jnp.where(lt2, ids16 - (OFF1 - 1), dummy1))
            loc2 = jnp.where(lt2, (ids16 >> 1) + 1, ids16 - (OFF2 - 1))
            sl = pl.ds(t * LANES, LANES)
            loc0_v[j, sl] = loc0
            loc1_v[j, sl] = loc1
            loc2_v[j, sl] = loc2
    cps = []
    for j in range(NCHUNK):
        dst = pl.ds(j * CHUNK, CHUNK)
        cps.append(pltpu.async_copy(t0_hbm.at[loc0_v.at[j]], r0_v.at[dst], gsem))
        cps.append(pltpu.async_copy(t1_hbm.at[loc1_v.at[j]], r1_v.at[dst], gsem))
        cps.append(pltpu.async_copy(t2_hbm.at[loc2_v.at[j]], r2_v.at[dst], gsem))
    for cp in cps:
        cp.wait()
    out_sl = pl.ds(base, BPW)
    wps = [
        pltpu.async_copy(r0_v, e0_hbm.at[out_sl], wsem),
        pltpu.async_copy(r1_v, e1_hbm.at[out_sl], wsem),
        pltpu.async_copy(r2_v, e2_hbm.at[out_sl], wsem),
    ]
    for wp in wps:
        wp.wait()


def _sc_gather(ids, T0, T1, T2):
    mesh = plsc.VectorSubcoreMesh(
        core_axis_name="c", subcore_axis_name="s",
        num_cores=NUM_CORES, num_subcores=NUM_SUBCORES)
    f = pl.kernel(
        _gather_body,
        out_type=(
            jax.ShapeDtypeStruct((BATCH, BASE_DIM), jnp.float32),
            jax.ShapeDtypeStruct((BATCH, D1), jnp.float32),
            jax.ShapeDtypeStruct((BATCH, D2), jnp.float32),
        ),
        mesh=mesh,
        compiler_params=pltpu.CompilerParams(use_tc_tiling_on_sc=False),
        scratch_types=[
            pltpu.VMEM((BPW,), jnp.int32),
            pltpu.VMEM((NCHUNK, CHUNK), jnp.int32),
            pltpu.VMEM((NCHUNK, CHUNK), jnp.int32),
            pltpu.VMEM((NCHUNK, CHUNK), jnp.int32),
            pltpu.VMEM((BPW, BASE_DIM), jnp.float32),
            pltpu.VMEM((BPW, D1), jnp.float32),
            pltpu.VMEM((BPW, D2), jnp.float32),
            pltpu.SemaphoreType.DMA,
            pltpu.SemaphoreType.DMA,
        ],
    )
    return f(ids, T0, T1, T2)


BT = 2048


def _proj_body(ids_ref, e0_ref, e1_ref, e2_ref, w1_ref, b1_ref, w2_ref, b2_ref,
               out_ref):
    ids = ids_ref[...]
    m0 = (ids < OFF1).astype(jnp.float32)
    m2 = (ids >= OFF2).astype(jnp.float32)
    m1 = (1.0 - m0) - m2
    p1 = jnp.dot(e1_ref[...], w1_ref[...],
                 preferred_element_type=jnp.float32) + b1_ref[...]
    p2 = jnp.dot(e2_ref[...], w2_ref[...],
                 preferred_element_type=jnp.float32) + b2_ref[...]
    out_ref[...] = e0_ref[...] * m0 + p1 * m1 + p2 * m2


def _tc_project(ids2d, e0, e1, e2, W1, b1, W2, b2):
    grid = (BATCH // BT,)
    return pl.pallas_call(
        _proj_body,
        grid=grid,
        in_specs=[
            pl.BlockSpec((BT, 1), lambda i: (i, 0)),
            pl.BlockSpec((BT, BASE_DIM), lambda i: (i, 0)),
            pl.BlockSpec((BT, D1), lambda i: (i, 0)),
            pl.BlockSpec((BT, D2), lambda i: (i, 0)),
            pl.BlockSpec((D1, BASE_DIM), lambda i: (0, 0)),
            pl.BlockSpec((1, BASE_DIM), lambda i: (0, 0)),
            pl.BlockSpec((D2, BASE_DIM), lambda i: (0, 0)),
            pl.BlockSpec((1, BASE_DIM), lambda i: (0, 0)),
        ],
        out_specs=pl.BlockSpec((BT, BASE_DIM), lambda i: (i, 0)),
        out_shape=jax.ShapeDtypeStruct((BATCH, BASE_DIM), jnp.float32),
    )(ids2d, e0, e1, e2, W1, b1, W2, b2)


def kernel(inputs, T0, T1, W1, b1, T2, W2, b2):
    ids = inputs.astype(jnp.int32)
    e0, e1, e2 = _sc_gather(ids, T0, T1, T2)
    return _tc_project(ids.reshape(BATCH, 1), e0, e1, e2,
                       W1, b1.reshape(1, BASE_DIM), W2, b2.reshape(1, BASE_DIM))


# speedup vs baseline: 2.9772x; 1.0489x over previous
import functools

import jax
import jax.numpy as jnp
from jax import lax
from jax.experimental import pallas as pl
from jax.experimental.pallas import tpu as pltpu
from jax.experimental.pallas import tpu_sc as plsc

BATCH = 16384
BASE_DIM = 64
D1 = 32
D2 = 16
OFF1 = 100000
OFF2 = 400000

NUM_CORES = 2
NUM_SUBCORES = 16
NW = NUM_CORES * NUM_SUBCORES
BPW = BATCH // NW
CHUNK = 128
NCHUNK = BPW // CHUNK
LANES = 16
PACK = 128


def _gather_body(ids_hbm, t0_hbm, t1_hbm, t2_hbm, e_hbm,
                 ids_v, loc0_v, loc1_v, loc2_v, r0_v, r1_v, r2_v, gsem, wsem):
    wid = lax.axis_index("s") * NUM_CORES + lax.axis_index("c")
    base = wid * BPW
    pltpu.sync_copy(ids_hbm.at[pl.ds(base, BPW)], ids_v)
    for j in range(NCHUNK):
        for t in range(CHUNK // LANES):
            off = j * CHUNK + t * LANES
            ids16 = ids_v[pl.ds(off, LANES)]
            lt1 = ids16 < OFF1
            lt2 = ids16 < OFF2
            loc0 = jnp.where(lt1, ids16 + 1, (ids16 >> 4) + 1)
            dummy1 = (ids16 >> 2) + 1
            loc1 = jnp.where(lt1, dummy1,
                             jnp.where(lt2, ids16 - (OFF1 - 1), dummy1))
            loc2 = jnp.where(lt2, (ids16 >> 1) + 1, ids16 - (OFF2 - 1))
            sl = pl.ds(t * LANES, LANES)
            loc0_v[j, sl] = loc0
            loc1_v[j, sl] = loc1
            loc2_v[j, sl] = loc2
    cps = []
    for j in range(NCHUNK):
        dst = pl.ds(j * CHUNK, CHUNK)
        cps.append(pltpu.async_copy(t0_hbm.at[loc0_v.at[j]], r0_v.at[dst], gsem))
        cps.append(pltpu.async_copy(t1_hbm.at[loc1_v.at[j]], r1_v.at[dst], gsem))
        cps.append(pltpu.async_copy(t2_hbm.at[loc2_v.at[j]], r2_v.at[dst], gsem))
    for cp in cps:
        cp.wait()
    rows = pl.ds(base, BPW)
    wps = [
        pltpu.async_copy(r0_v, e_hbm.at[rows, pl.ds(0, BASE_DIM)], wsem),
        pltpu.async_copy(r1_v, e_hbm.at[rows, pl.ds(BASE_DIM, D1)], wsem),
        pltpu.async_copy(r2_v, e_hbm.at[rows, pl.ds(BASE_DIM + D1, D2)], wsem),
    ]
    for wp in wps:
        wp.wait()


def _sc_gather(ids, T0, T1, T2):
    mesh = plsc.VectorSubcoreMesh(
        core_axis_name="c", subcore_axis_name="s",
        num_cores=NUM_CORES, num_subcores=NUM_SUBCORES)
    f = pl.kernel(
        _gather_body,
        out_type=jax.ShapeDtypeStruct((BATCH, PACK), jnp.float32),
        mesh=mesh,
        compiler_params=pltpu.CompilerParams(use_tc_tiling_on_sc=False),
        scratch_types=[
            pltpu.VMEM((BPW,), jnp.int32),
            pltpu.VMEM((NCHUNK, CHUNK), jnp.int32),
            pltpu.VMEM((NCHUNK, CHUNK), jnp.int32),
            pltpu.VMEM((NCHUNK, CHUNK), jnp.int32),
            pltpu.VMEM((BPW, BASE_DIM), jnp.float32),
            pltpu.VMEM((BPW, D1), jnp.float32),
            pltpu.VMEM((BPW, D2), jnp.float32),
            pltpu.SemaphoreType.DMA,
            pltpu.SemaphoreType.DMA,
        ],
    )
    return f(ids, T0, T1, T2)


BT = 2048


def _proj_body(ids_ref, e_ref, w1_ref, b1_ref, w2_ref, b2_ref, out_ref):
    ids = ids_ref[...]
    m0 = (ids < OFF1).astype(jnp.float32)
    m2 = (ids >= OFF2).astype(jnp.float32)
    m1 = (1.0 - m0) - m2
    e = e_ref[...]
    e0 = e[:, :BASE_DIM]
    e1 = e[:, BASE_DIM:BASE_DIM + D1]
    e2 = e[:, BASE_DIM + D1:BASE_DIM + D1 + D2]
    p1 = jnp.dot(e1, w1_ref[...], preferred_element_type=jnp.float32) + b1_ref[...]
    p2 = jnp.dot(e2, w2_ref[...], preferred_element_type=jnp.float32) + b2_ref[...]
    out_ref[...] = e0 * m0 + p1 * m1 + p2 * m2


def _tc_project(ids2d, e, W1, b1, W2, b2):
    grid = (BATCH // BT,)
    return pl.pallas_call(
        _proj_body,
        grid=grid,
        in_specs=[
            pl.BlockSpec((BT, 1), lambda i: (i, 0)),
            pl.BlockSpec((BT, PACK), lambda i: (i, 0)),
            pl.BlockSpec((D1, BASE_DIM), lambda i: (0, 0)),
            pl.BlockSpec((1, BASE_DIM), lambda i: (0, 0)),
            pl.BlockSpec((D2, BASE_DIM), lambda i: (0, 0)),
            pl.BlockSpec((1, BASE_DIM), lambda i: (0, 0)),
        ],
        out_specs=pl.BlockSpec((BT, BASE_DIM), lambda i: (i, 0)),
        out_shape=jax.ShapeDtypeStruct((BATCH, BASE_DIM), jnp.float32),
    )(ids2d, e, W1, b1, W2, b2)


def kernel(inputs, T0, T1, W1, b1, T2, W2, b2):
    ids = inputs.astype(jnp.int32)
    e = _sc_gather(ids, T0, T1, T2)
    return _tc_project(ids.reshape(BATCH, 1), e,
                       W1, b1.reshape(1, BASE_DIM), W2, b2.reshape(1, BASE_DIM))
